# baseline (speedup 1.0000x reference)
import functools

import jax
import jax.numpy as jnp
from jax import lax
from jax.experimental import pallas as pl
from jax.experimental.pallas import tpu as pltpu
from jax.experimental.pallas import tpu_sc as plsc

EMB = 64


def _emb_body(w_hbm, idx_hbm, out_hbm, idx_v, row_v, sem):
    cid = lax.axis_index("c")
    sid = lax.axis_index("s")

    @pl.when(jnp.logical_and(cid == 0, sid == 0))
    def _():
        pltpu.sync_copy(idx_hbm, idx_v)
        pltpu.async_copy(w_hbm.at[idx_v], row_v, sem).wait()
        pltpu.sync_copy(row_v, out_hbm)


def kernel(_image, _label, dataset_idx, weight):
    idx = jnp.reshape(dataset_idx, (1,)).astype(jnp.int32)
    mesh = plsc.VectorSubcoreMesh(core_axis_name="c", subcore_axis_name="s")
    emb = functools.partial(
        pl.kernel,
        mesh=mesh,
        out_type=jax.ShapeDtypeStruct((1, EMB), jnp.float32),
        scratch_types=[
            pltpu.VMEM((1,), jnp.int32),
            pltpu.VMEM((1, EMB), jnp.float32),
            pltpu.SemaphoreType.DMA,
        ],
        compiler_params=pltpu.CompilerParams(use_tc_tiling_on_sc=False),
    )(_emb_body)
    out = emb(weight, idx)
    return jnp.reshape(out, (EMB,))


# speedup vs baseline: 1.5224x; 1.5224x over previous
import functools

import jax
import jax.numpy as jnp
from jax import lax
from jax.experimental import pallas as pl
from jax.experimental.pallas import tpu as pltpu
from jax.experimental.pallas import tpu_sc as plsc

EMB = 64


def _emb_body(w_hbm, idx_hbm, out_hbm, idx_v, row_v, sem):
    cid = lax.axis_index("c")
    sid = lax.axis_index("s")

    @pl.when(jnp.logical_and(cid == 0, sid == 0))
    def _():
        pltpu.sync_copy(idx_hbm, idx_v)
        i = idx_v[...][0]
        pltpu.async_copy(w_hbm.at[pl.ds(i, 1)], row_v, sem).wait()
        pltpu.sync_copy(row_v, out_hbm)


def kernel(_image, _label, dataset_idx, weight):
    idx = jnp.broadcast_to(jnp.reshape(dataset_idx, (1,)).astype(jnp.int32), (16,))
    mesh = plsc.VectorSubcoreMesh(core_axis_name="c", subcore_axis_name="s")
    emb = functools.partial(
        pl.kernel,
        mesh=mesh,
        out_type=jax.ShapeDtypeStruct((1, EMB), jnp.float32),
        scratch_types=[
            pltpu.VMEM((16,), jnp.int32),
            pltpu.VMEM((1, EMB), jnp.float32),
            pltpu.SemaphoreType.DMA,
        ],
    )(_emb_body)
    out = emb(weight, idx)
    return jnp.reshape(out, (EMB,))


# speedup vs baseline: 1.5389x; 1.0109x over previous
import functools

import jax
import jax.numpy as jnp
from jax import lax
from jax.experimental import pallas as pl
from jax.experimental.pallas import tpu as pltpu
from jax.experimental.pallas import tpu_sc as plsc

EMB = 64


def _emb_body(w_hbm, idx_hbm, out_hbm, idx_v, row_v, sem):
    pltpu.sync_copy(idx_hbm, idx_v)
    i = idx_v[...][0]
    pltpu.async_copy(w_hbm.at[pl.ds(i, 1)], row_v, sem).wait()
    pltpu.sync_copy(row_v, out_hbm)


def kernel(_image, _label, dataset_idx, weight):
    idx = jnp.broadcast_to(jnp.reshape(dataset_idx, (1,)).astype(jnp.int32), (16,))
    mesh = plsc.VectorSubcoreMesh(
        core_axis_name="c", subcore_axis_name="s", num_cores=1, num_subcores=1
    )
    emb = functools.partial(
        pl.kernel,
        mesh=mesh,
        out_type=jax.ShapeDtypeStruct((1, EMB), jnp.float32),
        scratch_types=[
            pltpu.VMEM((16,), jnp.int32),
            pltpu.VMEM((1, EMB), jnp.float32),
            pltpu.SemaphoreType.DMA,
        ],
    )(_emb_body)
    out = emb(weight, idx)
    return jnp.reshape(out, (EMB,))


# speedup vs baseline: 1.5673x; 1.0184x over previous
import functools

import jax
import jax.numpy as jnp
from jax import lax
from jax.experimental import pallas as pl
from jax.experimental.pallas import tpu as pltpu
from jax.experimental.pallas import tpu_sc as plsc

EMB = 64


def _emb_body(w_hbm, idx_hbm, out_hbm, idx_v, row_v, sem):
    pltpu.sync_copy(idx_hbm, idx_v)
    i = idx_v[...][0]
    pltpu.async_copy(w_hbm.at[pl.ds(i, 1)], row_v, sem).wait()
    pltpu.sync_copy(row_v, out_hbm)


def kernel(_image, _label, dataset_idx, weight):
    idx = jnp.broadcast_to(jnp.reshape(dataset_idx, (1,)).astype(jnp.int32), (16,))
    mesh = plsc.VectorSubcoreMesh(
        core_axis_name="c", subcore_axis_name="s", num_cores=1, num_subcores=1
    )
    emb = functools.partial(
        pl.kernel,
        mesh=mesh,
        out_type=jax.ShapeDtypeStruct((1, EMB), jnp.float32),
        scratch_types=[
            pltpu.VMEM((16,), jnp.int32),
            pltpu.VMEM((1, EMB), jnp.float32),
            pltpu.SemaphoreType.DMA,
        ],
        compiler_params=pltpu.CompilerParams(
            skip_device_barrier=True,
            disable_bounds_checks=True,
            disable_semaphore_checks=True,
        ),
    )(_emb_body)
    out = emb(weight, idx)
    return jnp.reshape(out, (EMB,))


# speedup vs baseline: 1.6109x; 1.0278x over previous
import functools

import jax
import jax.numpy as jnp
from jax import lax
from jax.experimental import pallas as pl
from jax.experimental.pallas import tpu as pltpu
from jax.experimental.pallas import tpu_sc as plsc

EMB = 64


def _emb_body(w_hbm, idx_hbm, out_hbm, idx_s):
    pltpu.sync_copy(idx_hbm, idx_s)
    i = idx_s[0]
    pltpu.sync_copy(w_hbm.at[pl.ds(i, 1)], out_hbm)


def kernel(_image, _label, dataset_idx, weight):
    idx = jnp.reshape(dataset_idx, (1,)).astype(jnp.int32)
    mesh = plsc.ScalarSubcoreMesh(axis_name="c", num_cores=1)
    emb = functools.partial(
        pl.kernel,
        mesh=mesh,
        out_type=jax.ShapeDtypeStruct((1, EMB), jnp.float32),
        scratch_types=[
            pltpu.SMEM((1,), jnp.int32),
        ],
    )(_emb_body)
    out = emb(weight, idx)
    return jnp.reshape(out, (EMB,))


# speedup vs baseline: 1.6123x; 1.0009x over previous
import functools

import jax
import jax.numpy as jnp
from jax import lax
from jax.experimental import pallas as pl
from jax.experimental.pallas import tpu as pltpu
from jax.experimental.pallas import tpu_sc as plsc

EMB = 64


def _emb_body(w_hbm, idx_hbm, out_hbm, idx_s):
    pltpu.sync_copy(idx_hbm, idx_s)
    i = idx_s[0]
    pltpu.sync_copy(w_hbm.at[pl.ds(i, 1)], out_hbm)


def kernel(_image, _label, dataset_idx, weight):
    idx = jnp.reshape(dataset_idx, (1,)).astype(jnp.int32)
    mesh = plsc.ScalarSubcoreMesh(axis_name="c", num_cores=1)
    emb = functools.partial(
        pl.kernel,
        mesh=mesh,
        out_type=jax.ShapeDtypeStruct((1, EMB), jnp.float32),
        scratch_types=[
            pltpu.SMEM((1,), jnp.int32),
        ],
        compiler_params=pltpu.CompilerParams(use_tc_tiling_on_sc=True),
    )(_emb_body)
    out = emb(weight, idx)
    return jnp.reshape(out, (EMB,))
